# baseline (speedup 1.0000x reference)
import math

import jax
import jax.numpy as jnp
from jax import lax
from jax.experimental import pallas as pl
from jax.experimental.pallas import tpu as pltpu
from jax.experimental.pallas import tpu_sc as plsc

D = 128
B = 16
S = 4096
NC = 2
NS = 16
L = 16
NJ = D // L

NSC = 2
BPC = NSC // 2
TPS = NS // BPC
R = S // TPS
CH = min(R, 128)
NCHUNK = R // CH
PCOLS = 2 * D + 128

BS = 512
NSB = S // BS
NINF = -1e30


def _sc_body(in0_hbm, in1_hbm, w_hbm, fp_hbm, out_hbm,
             buf0, buf1, wmat, fp_ref, scores_ref, w_ref, stage, shared, part,
             fin, sem00, sem01, sem10, sem11):
    s_idx = lax.axis_index("s")
    c_idx = lax.axis_index("c")
    b = c_idx * BPC + s_idx // TPS
    p_idx = s_idx % TPS
    lane = lax.iota(jnp.int32, L)

    pltpu.sync_copy(w_hbm, wmat)
    pltpu.sync_copy(fp_hbm, fp_ref)
    inv_scale = 1.0 / math.sqrt(float(D))
    vs = []
    for j in range(NJ):
        diag_idx = lane + j * L
        dj = plsc.load_gather(wmat, [diag_idx, diag_idx])
        vs.append(dj * fp_ref[pl.ds(j * L, L)] * inv_scale)

    zeros = jnp.zeros((L,), jnp.float32)
    accs0 = tuple(zeros for _ in range(2 * NJ))
    row0 = p_idx * R
    sems0 = (sem00, sem01)
    sems1 = (sem10, sem11)

    def issue(c, slot):
        start = row0 + c * CH
        pltpu.async_copy(in1_hbm.at[b, pl.ds(start, CH)], buf1.at[slot],
                         sems1[slot])
        pltpu.async_copy(in0_hbm.at[b, pl.ds(start, CH)], buf0.at[slot],
                         sems0[slot])

    issue(0, 0)

    @pl.loop(0, NCHUNK // 2, init_carry=(jnp.float32(NINF), zeros, accs0))
    def chunk_loop(i, carry):
        m, l16, accs = carry
        for slot in range(2):
            c = 2 * i + slot

            @pl.when(c + 1 < NCHUNK)
            def _():
                issue(c + 1, 1 - slot)

            b1 = buf1.at[slot]
            b0 = buf0.at[slot]
            pltpu.make_async_copy(in1_hbm.at[b, pl.ds(row0, CH)],
                                  b1, sems1[slot]).wait()

            @pl.loop(0, CH // L)
            def score_groups(g):
                def row_score(r, sc16):
                    s = g * L + r
                    acc = vs[0] * b1[s, pl.ds(0, L)]
                    for j in range(1, NJ):
                        acc = acc + vs[j] * b1[s, pl.ds(j * L, L)]
                    sc = jnp.sum(acc)
                    return jnp.where(lane == r, sc, sc16)
                scores_ref[pl.ds(g * L, L)] = lax.fori_loop(
                    0, L, row_score, zeros, unroll=True)

            def group_max(g, m16):
                return jnp.maximum(m16, scores_ref[pl.ds(g * L, L)])
            m16 = lax.fori_loop(0, CH // L, group_max,
                                jnp.full((L,), NINF, jnp.float32),
                                unroll=4)
            mc = jnp.max(m16)
            m_new = jnp.maximum(m, mc)
            rs16 = jnp.exp(jnp.full((L,), m - m_new, jnp.float32))
            l16 = l16 * rs16
            accs = tuple(a * rs16 for a in accs)
            m_full = jnp.full((L,), m_new, jnp.float32)
            m = m_new

            def group_w(g, l16):
                w16 = jnp.exp(scores_ref[pl.ds(g * L, L)] - m_full)
                w_ref[pl.ds(g * L, L)] = w16
                return l16 + w16
            l16 = lax.fori_loop(0, CH // L, group_w, l16, unroll=4)

            pltpu.make_async_copy(in0_hbm.at[b, pl.ds(row0, CH)],
                                  b0, sems0[slot]).wait()

            def row_acc(s, accs):
                wspl = plsc.load_gather(
                    w_ref, [jnp.full((L,), s, jnp.int32)])
                new = []
                for j in range(NJ):
                    new.append(accs[j] + wspl * b0[s, pl.ds(j * L, L)])
                for j in range(NJ):
                    new.append(accs[NJ + j] + wspl * b1[s, pl.ds(j * L, L)])
                return tuple(new)
            accs = lax.fori_loop(0, CH, row_acc, accs, unroll=4)
        return m, l16, accs

    m, l16, accs = chunk_loop

    for j in range(2 * NJ):
        stage[pl.ds(j * L, L)] = accs[j]
    lsum = jnp.sum(l16)
    mf = jnp.full((L,), m, jnp.float32)
    lf = jnp.full((L,), lsum, jnp.float32)
    stage[pl.ds(2 * D, L)] = jnp.where(lane == 0, mf,
                                       jnp.where(lane == 1, lf, zeros))
    pltpu.sync_copy(stage, shared.at[s_idx])
    plsc.subcore_barrier()

    @pl.when(p_idx == 0)
    def _():
        gbase = s_idx
        pltpu.sync_copy(shared, part)
        macc = [jnp.full((L,), m, jnp.float32)]
        lacc = [jnp.full((L,), lsum, jnp.float32)]
        cur = [list(accs)]
        for t in range(1, TPS):
            row = gbase + t
            m_o = plsc.load_gather(
                part, [jnp.full((L,), row, jnp.int32),
                       jnp.full((L,), 2 * D, jnp.int32)])
            l_o = plsc.load_gather(
                part, [jnp.full((L,), row, jnp.int32),
                       jnp.full((L,), 2 * D + 1, jnp.int32)])
            mb = jnp.maximum(macc[0], m_o)
            cs = jnp.exp(macc[0] - mb)
            co = jnp.exp(m_o - mb)
            lacc[0] = lacc[0] * cs + l_o * co
            nxt = []
            for j in range(2 * NJ):
                acc_o = part[row, pl.ds(j * L, L)]
                nxt.append(cur[0][j] * cs + acc_o * co)
            cur[0] = nxt
            macc[0] = mb
        rinv = 1.0 / lacc[0]
        for j in range(2 * NJ):
            fin[pl.ds(j * L, L)] = cur[0][j] * rinv
        pltpu.sync_copy(fin, out_hbm.at[b])


def _tc_body(v_ref, in0_ref, in1_ref, out_ref, acc_ref, m_ref, l_ref):
    sb = pl.program_id(1)

    @pl.when(sb == 0)
    def _():
        m_ref[0] = jnp.float32(NINF)
        l_ref[0] = jnp.float32(0.0)
        acc_ref[...] = jnp.zeros_like(acc_ref)

    x0 = in0_ref[0]
    x1 = in1_ref[0]
    vrow = v_ref[...]
    sc = jnp.sum(x1 * vrow, axis=1, keepdims=True)
    mc = jnp.max(sc)
    m_old = m_ref[0]
    m_new = jnp.maximum(m_old, mc)
    alpha = jnp.exp(m_old - m_new)
    w = jnp.exp(sc - m_new)
    l_ref[0] = l_ref[0] * alpha + jnp.sum(w)
    m_ref[0] = m_new
    a0 = jax.lax.dot_general(w, x0, (((0,), (0,)), ((), ())),
                             precision=jax.lax.Precision.HIGHEST,
                             preferred_element_type=jnp.float32)
    a1 = jax.lax.dot_general(w, x1, (((0,), (0,)), ((), ())),
                             precision=jax.lax.Precision.HIGHEST,
                             preferred_element_type=jnp.float32)
    acc_ref[0:1, :] = acc_ref[0:1, :] * alpha + a0
    acc_ref[1:2, :] = acc_ref[1:2, :] * alpha + a1

    @pl.when(sb == NSB - 1)
    def _():
        rinv = 1.0 / l_ref[0]
        out_ref[0, 0:1, pl.ds(0, D)] = acc_ref[0:1, :] * rinv
        out_ref[0, 0:1, pl.ds(D, D)] = acc_ref[1:2, :] * rinv


@jax.jit
def kernel(inputs_0, inputs_1, W, b, fingerprint):
    mesh = plsc.VectorSubcoreMesh(
        core_axis_name="c", subcore_axis_name="s",
        num_cores=NC, num_subcores=NS)
    sc_call = pl.kernel(
        _sc_body,
        out_type=jax.ShapeDtypeStruct((NSC, 2 * D), jnp.float32),
        mesh=mesh,
        compiler_params=pltpu.CompilerParams(needs_layout_passes=False),
        scratch_types=[
            pltpu.VMEM((2, CH, D), jnp.float32),
            pltpu.VMEM((2, CH, D), jnp.float32),
            pltpu.VMEM((D, D), jnp.float32),
            pltpu.VMEM((D,), jnp.float32),
            pltpu.VMEM((CH,), jnp.float32),
            pltpu.VMEM((CH,), jnp.float32),
            pltpu.VMEM((PCOLS,), jnp.float32),
            pltpu.VMEM_SHARED((NS, PCOLS), jnp.float32),
            pltpu.VMEM((NS, PCOLS), jnp.float32),
            pltpu.VMEM((2 * D,), jnp.float32),
            pltpu.SemaphoreType.DMA,
            pltpu.SemaphoreType.DMA,
            pltpu.SemaphoreType.DMA,
            pltpu.SemaphoreType.DMA,
        ],
    )
    sc_out = sc_call(inputs_0, inputs_1, W, fingerprint)

    scale = math.sqrt(float(D))
    v = (jnp.diagonal(W) * fingerprint / scale).astype(jnp.float32)
    v = v.reshape(1, D)
    nb_tc = B - NSC
    tc_out = pl.pallas_call(
        _tc_body,
        grid=(nb_tc, NSB),
        in_specs=[
            pl.BlockSpec((1, D), lambda ib, sb: (0, 0)),
            pl.BlockSpec((1, BS, D), lambda ib, sb: (ib + NSC, sb, 0)),
            pl.BlockSpec((1, BS, D), lambda ib, sb: (ib + NSC, sb, 0)),
        ],
        out_specs=pl.BlockSpec((1, 1, 2 * D), lambda ib, sb: (ib, 0, 0)),
        out_shape=jax.ShapeDtypeStruct((nb_tc, 1, 2 * D), jnp.float32),
        scratch_shapes=[
            pltpu.VMEM((2, D), jnp.float32),
            pltpu.SMEM((1,), jnp.float32),
            pltpu.SMEM((1,), jnp.float32),
        ],
    )(v, inputs_0, inputs_1)

    return jnp.concatenate([sc_out, tc_out.reshape(nb_tc, 2 * D)], axis=0)


# speedup vs baseline: 1.0898x; 1.0898x over previous
import math

import jax
import jax.numpy as jnp
from jax import lax
from jax.experimental import pallas as pl
from jax.experimental.pallas import tpu as pltpu
from jax.experimental.pallas import tpu_sc as plsc

D = 128
B = 16
S = 4096
NC = 2
NS = 16
L = 16
NJ = D // L

NSC = 2
BPC = NSC // 2
TPS = NS // BPC
R = S // TPS
CH = min(R, 128)
NCHUNK = R // CH
PCOLS = 2 * D + 128

BS = 512
NSB = S // BS
NINF = -1e30


def _sc_body(in0_hbm, in1_hbm, w_hbm, fp_hbm, out_hbm,
             buf0, buf1, wmat, fp_ref, scores_ref, w_ref, stage, shared, part,
             fin, sem00, sem01, sem10, sem11):
    s_idx = lax.axis_index("s")
    c_idx = lax.axis_index("c")
    b = c_idx * BPC + s_idx // TPS
    p_idx = s_idx % TPS
    lane = lax.iota(jnp.int32, L)

    pltpu.sync_copy(w_hbm, wmat)
    pltpu.sync_copy(fp_hbm, fp_ref)
    inv_scale = 1.0 / math.sqrt(float(D))
    vs = []
    for j in range(NJ):
        diag_idx = lane + j * L
        dj = plsc.load_gather(wmat, [diag_idx, diag_idx])
        vs.append(dj * fp_ref[pl.ds(j * L, L)] * inv_scale)

    zeros = jnp.zeros((L,), jnp.float32)
    accs0 = tuple(zeros for _ in range(2 * NJ))
    row0 = p_idx * R
    sems0 = (sem00, sem01)
    sems1 = (sem10, sem11)

    def issue(c, slot):
        start = row0 + c * CH
        pltpu.async_copy(in1_hbm.at[b, pl.ds(start, CH)], buf1.at[slot],
                         sems1[slot])
        pltpu.async_copy(in0_hbm.at[b, pl.ds(start, CH)], buf0.at[slot],
                         sems0[slot])

    issue(0, 0)

    @pl.loop(0, NCHUNK // 2, init_carry=(jnp.float32(NINF), zeros, accs0))
    def chunk_loop(i, carry):
        m, l16, accs = carry
        for slot in range(2):
            c = 2 * i + slot

            @pl.when(c + 1 < NCHUNK)
            def _():
                issue(c + 1, 1 - slot)

            b1 = buf1.at[slot]
            b0 = buf0.at[slot]
            pltpu.make_async_copy(in1_hbm.at[b, pl.ds(row0, CH)],
                                  b1, sems1[slot]).wait()

            @pl.loop(0, CH // L)
            def score_groups(g):
                def row_score(r, sc16):
                    s = g * L + r
                    acc = vs[0] * b1[s, pl.ds(0, L)]
                    for j in range(1, NJ):
                        acc = acc + vs[j] * b1[s, pl.ds(j * L, L)]
                    sc = jnp.sum(acc)
                    return jnp.where(lane == r, sc, sc16)
                scores_ref[pl.ds(g * L, L)] = lax.fori_loop(
                    0, L, row_score, zeros, unroll=True)

            def group_max(g, m16):
                return jnp.maximum(m16, scores_ref[pl.ds(g * L, L)])
            m16 = lax.fori_loop(0, CH // L, group_max,
                                jnp.full((L,), NINF, jnp.float32),
                                unroll=4)
            mc = jnp.max(m16)
            m_new = jnp.maximum(m, mc)
            rs16 = jnp.exp(jnp.full((L,), m - m_new, jnp.float32))
            l16 = l16 * rs16
            accs = tuple(a * rs16 for a in accs)
            m_full = jnp.full((L,), m_new, jnp.float32)
            m = m_new

            def group_w(g, l16):
                w16 = jnp.exp(scores_ref[pl.ds(g * L, L)] - m_full)
                w_ref[pl.ds(g * L, L)] = w16
                return l16 + w16
            l16 = lax.fori_loop(0, CH // L, group_w, l16, unroll=4)

            pltpu.make_async_copy(in0_hbm.at[b, pl.ds(row0, CH)],
                                  b0, sems0[slot]).wait()

            def row_acc(s, accs):
                wspl = plsc.load_gather(
                    w_ref, [jnp.full((L,), s, jnp.int32)])
                new = []
                for j in range(NJ):
                    new.append(accs[j] + wspl * b0[s, pl.ds(j * L, L)])
                for j in range(NJ):
                    new.append(accs[NJ + j] + wspl * b1[s, pl.ds(j * L, L)])
                return tuple(new)
            accs = lax.fori_loop(0, CH, row_acc, accs, unroll=4)
        return m, l16, accs

    m, l16, accs = chunk_loop

    for j in range(2 * NJ):
        stage[pl.ds(j * L, L)] = accs[j]
    lsum = jnp.sum(l16)
    mf = jnp.full((L,), m, jnp.float32)
    lf = jnp.full((L,), lsum, jnp.float32)
    stage[pl.ds(2 * D, L)] = jnp.where(lane == 0, mf,
                                       jnp.where(lane == 1, lf, zeros))
    pltpu.sync_copy(stage, shared.at[s_idx])
    plsc.subcore_barrier()

    @pl.when(p_idx == 0)
    def _():
        gbase = s_idx
        pltpu.sync_copy(shared, part)
        macc = [jnp.full((L,), m, jnp.float32)]
        lacc = [jnp.full((L,), lsum, jnp.float32)]
        cur = [list(accs)]
        for t in range(1, TPS):
            row = gbase + t
            m_o = plsc.load_gather(
                part, [jnp.full((L,), row, jnp.int32),
                       jnp.full((L,), 2 * D, jnp.int32)])
            l_o = plsc.load_gather(
                part, [jnp.full((L,), row, jnp.int32),
                       jnp.full((L,), 2 * D + 1, jnp.int32)])
            mb = jnp.maximum(macc[0], m_o)
            cs = jnp.exp(macc[0] - mb)
            co = jnp.exp(m_o - mb)
            lacc[0] = lacc[0] * cs + l_o * co
            nxt = []
            for j in range(2 * NJ):
                acc_o = part[row, pl.ds(j * L, L)]
                nxt.append(cur[0][j] * cs + acc_o * co)
            cur[0] = nxt
            macc[0] = mb
        rinv = 1.0 / lacc[0]
        for j in range(2 * NJ):
            fin[pl.ds(j * L, L)] = cur[0][j] * rinv
        pltpu.sync_copy(fin, out_hbm.at[b])


def _tc_body(v_ref, in0_ref, in1_ref, out_ref, acc_ref, m_ref, l_ref):
    sb = pl.program_id(1)

    @pl.when(sb == 0)
    def _():
        m_ref[0] = jnp.float32(NINF)
        l_ref[0] = jnp.float32(0.0)
        acc_ref[...] = jnp.zeros_like(acc_ref)

    x0 = in0_ref[0]
    x1 = in1_ref[0]
    vrow = v_ref[...]
    sc = jnp.sum(x1 * vrow, axis=1, keepdims=True)
    mc = jnp.max(sc)
    m_old = m_ref[0]
    m_new = jnp.maximum(m_old, mc)
    alpha = jnp.exp(m_old - m_new)
    w = jnp.exp(sc - m_new)
    l_ref[0] = l_ref[0] * alpha + jnp.sum(w)
    m_ref[0] = m_new
    a0 = jnp.sum(w * x0, axis=0, keepdims=True)
    a1 = jnp.sum(w * x1, axis=0, keepdims=True)
    acc_ref[0:1, :] = acc_ref[0:1, :] * alpha + a0
    acc_ref[1:2, :] = acc_ref[1:2, :] * alpha + a1

    @pl.when(sb == NSB - 1)
    def _():
        rinv = 1.0 / l_ref[0]
        out_ref[0, 0:1, pl.ds(0, D)] = acc_ref[0:1, :] * rinv
        out_ref[0, 0:1, pl.ds(D, D)] = acc_ref[1:2, :] * rinv


@jax.jit
def kernel(inputs_0, inputs_1, W, b, fingerprint):
    mesh = plsc.VectorSubcoreMesh(
        core_axis_name="c", subcore_axis_name="s",
        num_cores=NC, num_subcores=NS)
    sc_call = pl.kernel(
        _sc_body,
        out_type=jax.ShapeDtypeStruct((NSC, 2 * D), jnp.float32),
        mesh=mesh,
        compiler_params=pltpu.CompilerParams(needs_layout_passes=False),
        scratch_types=[
            pltpu.VMEM((2, CH, D), jnp.float32),
            pltpu.VMEM((2, CH, D), jnp.float32),
            pltpu.VMEM((D, D), jnp.float32),
            pltpu.VMEM((D,), jnp.float32),
            pltpu.VMEM((CH,), jnp.float32),
            pltpu.VMEM((CH,), jnp.float32),
            pltpu.VMEM((PCOLS,), jnp.float32),
            pltpu.VMEM_SHARED((NS, PCOLS), jnp.float32),
            pltpu.VMEM((NS, PCOLS), jnp.float32),
            pltpu.VMEM((2 * D,), jnp.float32),
            pltpu.SemaphoreType.DMA,
            pltpu.SemaphoreType.DMA,
            pltpu.SemaphoreType.DMA,
            pltpu.SemaphoreType.DMA,
        ],
    )
    sc_out = sc_call(inputs_0, inputs_1, W, fingerprint)

    scale = math.sqrt(float(D))
    v = (jnp.diagonal(W) * fingerprint / scale).astype(jnp.float32)
    v = v.reshape(1, D)
    nb_tc = B - NSC
    tc_out = pl.pallas_call(
        _tc_body,
        grid=(nb_tc, NSB),
        in_specs=[
            pl.BlockSpec((1, D), lambda ib, sb: (0, 0)),
            pl.BlockSpec((1, BS, D), lambda ib, sb: (ib + NSC, sb, 0)),
            pl.BlockSpec((1, BS, D), lambda ib, sb: (ib + NSC, sb, 0)),
        ],
        out_specs=pl.BlockSpec((1, 1, 2 * D), lambda ib, sb: (ib, 0, 0)),
        out_shape=jax.ShapeDtypeStruct((nb_tc, 1, 2 * D), jnp.float32),
        scratch_shapes=[
            pltpu.VMEM((2, D), jnp.float32),
            pltpu.SMEM((1,), jnp.float32),
            pltpu.SMEM((1,), jnp.float32),
        ],
    )(v, inputs_0, inputs_1)

    return jnp.concatenate([sc_out, tc_out.reshape(nb_tc, 2 * D)], axis=0)


# speedup vs baseline: 1.4583x; 1.3381x over previous
import math

import jax
import jax.numpy as jnp
from jax import lax
from jax.experimental import pallas as pl
from jax.experimental.pallas import tpu as pltpu
from jax.experimental.pallas import tpu_sc as plsc

D = 128
B = 16
S = 4096
NC = 2
NS = 16
L = 16
NJ = D // L

NSC = 2
BPC = NSC // 2
TPS = NS // BPC
R = S // TPS
CH = min(R, 128)
NCHUNK = R // CH
PCOLS = 2 * D + 128

BS = 512
NSB = S // BS
TCB = 2
NINF = -1e30


def _sc_body(in0_hbm, in1_hbm, w_hbm, fp_hbm, out_hbm,
             buf0, buf1, wmat, fp_ref, scores_ref, w_ref, stage, shared, part,
             fin, sem00, sem01, sem10, sem11):
    s_idx = lax.axis_index("s")
    c_idx = lax.axis_index("c")
    b = c_idx * BPC + s_idx // TPS
    p_idx = s_idx % TPS
    lane = lax.iota(jnp.int32, L)

    pltpu.sync_copy(w_hbm, wmat)
    pltpu.sync_copy(fp_hbm, fp_ref)
    inv_scale = 1.0 / math.sqrt(float(D))
    vs = []
    for j in range(NJ):
        diag_idx = lane + j * L
        dj = plsc.load_gather(wmat, [diag_idx, diag_idx])
        vs.append(dj * fp_ref[pl.ds(j * L, L)] * inv_scale)

    zeros = jnp.zeros((L,), jnp.float32)
    accs0 = tuple(zeros for _ in range(2 * NJ))
    row0 = p_idx * R
    sems0 = (sem00, sem01)
    sems1 = (sem10, sem11)

    def issue(c, slot):
        start = row0 + c * CH
        pltpu.async_copy(in1_hbm.at[b, pl.ds(start, CH)], buf1.at[slot],
                         sems1[slot])
        pltpu.async_copy(in0_hbm.at[b, pl.ds(start, CH)], buf0.at[slot],
                         sems0[slot])

    issue(0, 0)

    @pl.loop(0, NCHUNK // 2, init_carry=(jnp.float32(NINF), zeros, accs0))
    def chunk_loop(i, carry):
        m, l16, accs = carry
        for slot in range(2):
            c = 2 * i + slot

            @pl.when(c + 1 < NCHUNK)
            def _():
                issue(c + 1, 1 - slot)

            b1 = buf1.at[slot]
            b0 = buf0.at[slot]
            pltpu.make_async_copy(in1_hbm.at[b, pl.ds(row0, CH)],
                                  b1, sems1[slot]).wait()

            @pl.loop(0, CH // L)
            def score_groups(g):
                def row_score(r, sc16):
                    s = g * L + r
                    acc = vs[0] * b1[s, pl.ds(0, L)]
                    for j in range(1, NJ):
                        acc = acc + vs[j] * b1[s, pl.ds(j * L, L)]
                    sc = jnp.sum(acc)
                    return jnp.where(lane == r, sc, sc16)
                scores_ref[pl.ds(g * L, L)] = lax.fori_loop(
                    0, L, row_score, zeros, unroll=True)

            def group_max(g, m16):
                return jnp.maximum(m16, scores_ref[pl.ds(g * L, L)])
            m16 = lax.fori_loop(0, CH // L, group_max,
                                jnp.full((L,), NINF, jnp.float32),
                                unroll=4)
            mc = jnp.max(m16)
            m_new = jnp.maximum(m, mc)
            rs16 = jnp.exp(jnp.full((L,), m - m_new, jnp.float32))
            l16 = l16 * rs16
            accs = tuple(a * rs16 for a in accs)
            m_full = jnp.full((L,), m_new, jnp.float32)
            m = m_new

            def group_w(g, l16):
                w16 = jnp.exp(scores_ref[pl.ds(g * L, L)] - m_full)
                w_ref[pl.ds(g * L, L)] = w16
                return l16 + w16
            l16 = lax.fori_loop(0, CH // L, group_w, l16, unroll=4)

            pltpu.make_async_copy(in0_hbm.at[b, pl.ds(row0, CH)],
                                  b0, sems0[slot]).wait()

            def row_acc(s, accs):
                wspl = plsc.load_gather(
                    w_ref, [jnp.full((L,), s, jnp.int32)])
                new = []
                for j in range(NJ):
                    new.append(accs[j] + wspl * b0[s, pl.ds(j * L, L)])
                for j in range(NJ):
                    new.append(accs[NJ + j] + wspl * b1[s, pl.ds(j * L, L)])
                return tuple(new)
            accs = lax.fori_loop(0, CH, row_acc, accs, unroll=4)
        return m, l16, accs

    m, l16, accs = chunk_loop

    for j in range(2 * NJ):
        stage[pl.ds(j * L, L)] = accs[j]
    lsum = jnp.sum(l16)
    mf = jnp.full((L,), m, jnp.float32)
    lf = jnp.full((L,), lsum, jnp.float32)
    stage[pl.ds(2 * D, L)] = jnp.where(lane == 0, mf,
                                       jnp.where(lane == 1, lf, zeros))
    pltpu.sync_copy(stage, shared.at[s_idx])
    plsc.subcore_barrier()

    @pl.when(p_idx == 0)
    def _():
        gbase = s_idx
        pltpu.sync_copy(shared, part)
        macc = [jnp.full((L,), m, jnp.float32)]
        lacc = [jnp.full((L,), lsum, jnp.float32)]
        cur = [list(accs)]
        for t in range(1, TPS):
            row = gbase + t
            m_o = plsc.load_gather(
                part, [jnp.full((L,), row, jnp.int32),
                       jnp.full((L,), 2 * D, jnp.int32)])
            l_o = plsc.load_gather(
                part, [jnp.full((L,), row, jnp.int32),
                       jnp.full((L,), 2 * D + 1, jnp.int32)])
            mb = jnp.maximum(macc[0], m_o)
            cs = jnp.exp(macc[0] - mb)
            co = jnp.exp(m_o - mb)
            lacc[0] = lacc[0] * cs + l_o * co
            nxt = []
            for j in range(2 * NJ):
                acc_o = part[row, pl.ds(j * L, L)]
                nxt.append(cur[0][j] * cs + acc_o * co)
            cur[0] = nxt
            macc[0] = mb
        rinv = 1.0 / lacc[0]
        for j in range(2 * NJ):
            fin[pl.ds(j * L, L)] = cur[0][j] * rinv
        pltpu.sync_copy(fin, out_hbm.at[b])


def _fold8(y):
    n = y.shape[0]
    while n > 8:
        n //= 2
        y2 = y.reshape(2, n, D)
        y = y2[0] + y2[1]
    return y


def _tc_body(v_ref, in0_ref, in1_ref, out_ref, acc_ref, m_ref, l_ref):
    sb = pl.program_id(1)

    @pl.when(sb == 0)
    def _():
        m_ref[0] = jnp.float32(NINF)
        m_ref[1] = jnp.float32(NINF)
        l_ref[0] = jnp.float32(0.0)
        l_ref[1] = jnp.float32(0.0)
        acc_ref[...] = jnp.zeros_like(acc_ref)

    vrow = v_ref[...]
    for i in range(TCB):
        x0 = in0_ref[i]
        x1 = in1_ref[i]
        sc = jax.lax.dot_general(vrow, x1, (((1,), (1,)), ((), ())),
                                 preferred_element_type=jnp.float32)
        mc = jnp.max(sc)
        m_old = m_ref[i]
        m_new = jnp.maximum(m_old, mc)
        alpha = jnp.exp(m_old - m_new)
        w = jnp.exp(sc - m_new)
        l_ref[i] = l_ref[i] * alpha + jnp.sum(w)
        m_ref[i] = m_new
        w_hi = w.astype(jnp.bfloat16).astype(jnp.float32)
        w_lo = w - w_hi
        r0 = (jax.lax.dot_general(w_hi, x0, (((1,), (0,)), ((), ())),
                                  preferred_element_type=jnp.float32) +
              jax.lax.dot_general(w_lo, x0, (((1,), (0,)), ((), ())),
                                  preferred_element_type=jnp.float32))
        r1 = (jax.lax.dot_general(w_hi, x1, (((1,), (0,)), ((), ())),
                                  preferred_element_type=jnp.float32) +
              jax.lax.dot_general(w_lo, x1, (((1,), (0,)), ((), ())),
                                  preferred_element_type=jnp.float32))
        a = i * 2
        acc_ref[a:a + 1, :] = acc_ref[a:a + 1, :] * alpha + r0
        acc_ref[a + 1:a + 2, :] = acc_ref[a + 1:a + 2, :] * alpha + r1

    @pl.when(sb == NSB - 1)
    def _():
        for i in range(TCB):
            rinv = 1.0 / l_ref[i]
            a = i * 2
            out_ref[i, 0:1, pl.ds(0, D)] = acc_ref[a:a + 1, :] * rinv
            out_ref[i, 0:1, pl.ds(D, D)] = acc_ref[a + 1:a + 2, :] * rinv


@jax.jit
def kernel(inputs_0, inputs_1, W, b, fingerprint):
    mesh = plsc.VectorSubcoreMesh(
        core_axis_name="c", subcore_axis_name="s",
        num_cores=NC, num_subcores=NS)
    sc_call = pl.kernel(
        _sc_body,
        out_type=jax.ShapeDtypeStruct((NSC, 2 * D), jnp.float32),
        mesh=mesh,
        compiler_params=pltpu.CompilerParams(needs_layout_passes=False),
        scratch_types=[
            pltpu.VMEM((2, CH, D), jnp.float32),
            pltpu.VMEM((2, CH, D), jnp.float32),
            pltpu.VMEM((D, D), jnp.float32),
            pltpu.VMEM((D,), jnp.float32),
            pltpu.VMEM((CH,), jnp.float32),
            pltpu.VMEM((CH,), jnp.float32),
            pltpu.VMEM((PCOLS,), jnp.float32),
            pltpu.VMEM_SHARED((NS, PCOLS), jnp.float32),
            pltpu.VMEM((NS, PCOLS), jnp.float32),
            pltpu.VMEM((2 * D,), jnp.float32),
            pltpu.SemaphoreType.DMA,
            pltpu.SemaphoreType.DMA,
            pltpu.SemaphoreType.DMA,
            pltpu.SemaphoreType.DMA,
        ],
    )
    sc_out = sc_call(inputs_0, inputs_1, W, fingerprint)

    scale = math.sqrt(float(D))
    v = (jnp.diagonal(W) * fingerprint / scale).astype(jnp.float32)
    v = v.reshape(1, D)
    nb_tc = B - NSC
    tc_out = pl.pallas_call(
        _tc_body,
        grid=(nb_tc // TCB, NSB),
        in_specs=[
            pl.BlockSpec((1, D), lambda ib, sb: (0, 0)),
            pl.BlockSpec((TCB, BS, D), lambda ib, sb: (ib + NSC // TCB, sb, 0)),
            pl.BlockSpec((TCB, BS, D), lambda ib, sb: (ib + NSC // TCB, sb, 0)),
        ],
        out_specs=pl.BlockSpec((TCB, 1, 2 * D), lambda ib, sb: (ib, 0, 0)),
        out_shape=jax.ShapeDtypeStruct((nb_tc, 1, 2 * D), jnp.float32),
        scratch_shapes=[
            pltpu.VMEM((2 * TCB, D), jnp.float32),
            pltpu.SMEM((TCB,), jnp.float32),
            pltpu.SMEM((TCB,), jnp.float32),
        ],
    )(v, inputs_0, inputs_1)

    return jnp.concatenate([sc_out, tc_out.reshape(nb_tc, 2 * D)], axis=0)


# speedup vs baseline: 1.7697x; 1.2136x over previous
import math

import jax
import jax.numpy as jnp
from jax import lax
from jax.experimental import pallas as pl
from jax.experimental.pallas import tpu as pltpu
from jax.experimental.pallas import tpu_sc as plsc

D = 128
B = 16
S = 4096
NC = 2
NS = 16
L = 16
NJ = D // L

NSC = 16
BPC = NSC // 2
TPS = NS // BPC
R = S // TPS
CH = min(R, 128)
NCHUNK = R // CH
PCOLS = 2 * D + 128

BS = 512
NSB = S // BS
TCB = 2
NINF = -1e30


def _sc_body(in0_hbm, in1_hbm, w_hbm, fp_hbm, out_hbm,
             buf0, buf1, wmat, fp_ref, scores_ref, w_ref, stage, shared, part,
             fin, sem00, sem01, sem10, sem11):
    s_idx = lax.axis_index("s")
    c_idx = lax.axis_index("c")
    b = c_idx * BPC + s_idx // TPS
    p_idx = s_idx % TPS
    lane = lax.iota(jnp.int32, L)

    pltpu.sync_copy(w_hbm, wmat)
    pltpu.sync_copy(fp_hbm, fp_ref)
    inv_scale = 1.0 / math.sqrt(float(D))
    vs = []
    for j in range(NJ):
        diag_idx = lane + j * L
        dj = plsc.load_gather(wmat, [diag_idx, diag_idx])
        vs.append(dj * fp_ref[pl.ds(j * L, L)] * inv_scale)

    zeros = jnp.zeros((L,), jnp.float32)
    accs0 = tuple(zeros for _ in range(2 * NJ))
    row0 = p_idx * R
    sems0 = (sem00, sem01)
    sems1 = (sem10, sem11)

    def issue(c, slot):
        start = row0 + c * CH
        pltpu.async_copy(in1_hbm.at[b, pl.ds(start, CH)], buf1.at[slot],
                         sems1[slot])
        pltpu.async_copy(in0_hbm.at[b, pl.ds(start, CH)], buf0.at[slot],
                         sems0[slot])

    issue(0, 0)

    @pl.loop(0, NCHUNK // 2, init_carry=(jnp.float32(NINF), zeros, accs0))
    def chunk_loop(i, carry):
        m, l16, accs = carry
        for slot in range(2):
            c = 2 * i + slot

            @pl.when(c + 1 < NCHUNK)
            def _():
                issue(c + 1, 1 - slot)

            b1 = buf1.at[slot]
            b0 = buf0.at[slot]
            pltpu.make_async_copy(in1_hbm.at[b, pl.ds(row0, CH)],
                                  b1, sems1[slot]).wait()

            @pl.loop(0, CH // L)
            def score_groups(g):
                def row_score(r, sc16):
                    s = g * L + r
                    acc = vs[0] * b1[s, pl.ds(0, L)]
                    for j in range(1, NJ):
                        acc = acc + vs[j] * b1[s, pl.ds(j * L, L)]
                    sc = jnp.sum(acc)
                    return jnp.where(lane == r, sc, sc16)
                scores_ref[pl.ds(g * L, L)] = lax.fori_loop(
                    0, L, row_score, zeros, unroll=True)

            def group_max(g, m16):
                return jnp.maximum(m16, scores_ref[pl.ds(g * L, L)])
            m16 = lax.fori_loop(0, CH // L, group_max,
                                jnp.full((L,), NINF, jnp.float32),
                                unroll=4)
            mc = jnp.max(m16)
            m_new = jnp.maximum(m, mc)
            rs16 = jnp.exp(jnp.full((L,), m - m_new, jnp.float32))
            l16 = l16 * rs16
            accs = tuple(a * rs16 for a in accs)
            m_full = jnp.full((L,), m_new, jnp.float32)
            m = m_new

            def group_w(g, l16):
                w16 = jnp.exp(scores_ref[pl.ds(g * L, L)] - m_full)
                w_ref[pl.ds(g * L, L)] = w16
                return l16 + w16
            l16 = lax.fori_loop(0, CH // L, group_w, l16, unroll=4)

            pltpu.make_async_copy(in0_hbm.at[b, pl.ds(row0, CH)],
                                  b0, sems0[slot]).wait()

            def row_acc(s, accs):
                wspl = plsc.load_gather(
                    w_ref, [jnp.full((L,), s, jnp.int32)])
                new = []
                for j in range(NJ):
                    new.append(accs[j] + wspl * b0[s, pl.ds(j * L, L)])
                for j in range(NJ):
                    new.append(accs[NJ + j] + wspl * b1[s, pl.ds(j * L, L)])
                return tuple(new)
            accs = lax.fori_loop(0, CH, row_acc, accs, unroll=4)
        return m, l16, accs

    m, l16, accs = chunk_loop

    for j in range(2 * NJ):
        stage[pl.ds(j * L, L)] = accs[j]
    lsum = jnp.sum(l16)
    mf = jnp.full((L,), m, jnp.float32)
    lf = jnp.full((L,), lsum, jnp.float32)
    stage[pl.ds(2 * D, L)] = jnp.where(lane == 0, mf,
                                       jnp.where(lane == 1, lf, zeros))
    pltpu.sync_copy(stage, shared.at[s_idx])
    plsc.subcore_barrier()

    @pl.when(p_idx == 0)
    def _():
        gbase = s_idx
        pltpu.sync_copy(shared, part)
        macc = [jnp.full((L,), m, jnp.float32)]
        lacc = [jnp.full((L,), lsum, jnp.float32)]
        cur = [list(accs)]
        for t in range(1, TPS):
            row = gbase + t
            m_o = plsc.load_gather(
                part, [jnp.full((L,), row, jnp.int32),
                       jnp.full((L,), 2 * D, jnp.int32)])
            l_o = plsc.load_gather(
                part, [jnp.full((L,), row, jnp.int32),
                       jnp.full((L,), 2 * D + 1, jnp.int32)])
            mb = jnp.maximum(macc[0], m_o)
            cs = jnp.exp(macc[0] - mb)
            co = jnp.exp(m_o - mb)
            lacc[0] = lacc[0] * cs + l_o * co
            nxt = []
            for j in range(2 * NJ):
                acc_o = part[row, pl.ds(j * L, L)]
                nxt.append(cur[0][j] * cs + acc_o * co)
            cur[0] = nxt
            macc[0] = mb
        rinv = 1.0 / lacc[0]
        for j in range(2 * NJ):
            fin[pl.ds(j * L, L)] = cur[0][j] * rinv
        pltpu.sync_copy(fin, out_hbm.at[b])


def _fold8(y):
    n = y.shape[0]
    while n > 8:
        n //= 2
        y2 = y.reshape(2, n, D)
        y = y2[0] + y2[1]
    return y


def _tc_body(v_ref, in0_ref, in1_ref, out_ref, acc_ref, m_ref, l_ref):
    sb = pl.program_id(1)

    @pl.when(sb == 0)
    def _():
        m_ref[0] = jnp.float32(NINF)
        m_ref[1] = jnp.float32(NINF)
        l_ref[0] = jnp.float32(0.0)
        l_ref[1] = jnp.float32(0.0)
        acc_ref[...] = jnp.zeros_like(acc_ref)

    vrow = v_ref[...]
    for i in range(TCB):
        x0 = in0_ref[i]
        x1 = in1_ref[i]
        sc = jax.lax.dot_general(vrow, x1, (((1,), (1,)), ((), ())),
                                 preferred_element_type=jnp.float32)
        mc = jnp.max(sc)
        m_old = m_ref[i]
        m_new = jnp.maximum(m_old, mc)
        alpha = jnp.exp(m_old - m_new)
        w = jnp.exp(sc - m_new)
        l_ref[i] = l_ref[i] * alpha + jnp.sum(w)
        m_ref[i] = m_new
        w_hi = w.astype(jnp.bfloat16).astype(jnp.float32)
        w_lo = w - w_hi
        r0 = (jax.lax.dot_general(w_hi, x0, (((1,), (0,)), ((), ())),
                                  preferred_element_type=jnp.float32) +
              jax.lax.dot_general(w_lo, x0, (((1,), (0,)), ((), ())),
                                  preferred_element_type=jnp.float32))
        r1 = (jax.lax.dot_general(w_hi, x1, (((1,), (0,)), ((), ())),
                                  preferred_element_type=jnp.float32) +
              jax.lax.dot_general(w_lo, x1, (((1,), (0,)), ((), ())),
                                  preferred_element_type=jnp.float32))
        a = i * 2
        acc_ref[a:a + 1, :] = acc_ref[a:a + 1, :] * alpha + r0
        acc_ref[a + 1:a + 2, :] = acc_ref[a + 1:a + 2, :] * alpha + r1

    @pl.when(sb == NSB - 1)
    def _():
        for i in range(TCB):
            rinv = 1.0 / l_ref[i]
            a = i * 2
            out_ref[i, 0:1, pl.ds(0, D)] = acc_ref[a:a + 1, :] * rinv
            out_ref[i, 0:1, pl.ds(D, D)] = acc_ref[a + 1:a + 2, :] * rinv


@jax.jit
def kernel(inputs_0, inputs_1, W, b, fingerprint):
    mesh = plsc.VectorSubcoreMesh(
        core_axis_name="c", subcore_axis_name="s",
        num_cores=NC, num_subcores=NS)
    sc_call = pl.kernel(
        _sc_body,
        out_type=jax.ShapeDtypeStruct((NSC, 2 * D), jnp.float32),
        mesh=mesh,
        compiler_params=pltpu.CompilerParams(needs_layout_passes=False),
        scratch_types=[
            pltpu.VMEM((2, CH, D), jnp.float32),
            pltpu.VMEM((2, CH, D), jnp.float32),
            pltpu.VMEM((D, D), jnp.float32),
            pltpu.VMEM((D,), jnp.float32),
            pltpu.VMEM((CH,), jnp.float32),
            pltpu.VMEM((CH,), jnp.float32),
            pltpu.VMEM((PCOLS,), jnp.float32),
            pltpu.VMEM_SHARED((NS, PCOLS), jnp.float32),
            pltpu.VMEM((NS, PCOLS), jnp.float32),
            pltpu.VMEM((2 * D,), jnp.float32),
            pltpu.SemaphoreType.DMA,
            pltpu.SemaphoreType.DMA,
            pltpu.SemaphoreType.DMA,
            pltpu.SemaphoreType.DMA,
        ],
    )
    sc_out = sc_call(inputs_0, inputs_1, W, fingerprint)

    scale = math.sqrt(float(D))
    v = (jnp.diagonal(W) * fingerprint / scale).astype(jnp.float32)
    v = v.reshape(1, D)
    nb_tc = B - NSC
    if nb_tc == 0:
        return sc_out
    tc_out = pl.pallas_call(
        _tc_body,
        grid=(nb_tc // TCB, NSB),
        in_specs=[
            pl.BlockSpec((1, D), lambda ib, sb: (0, 0)),
            pl.BlockSpec((TCB, BS, D), lambda ib, sb: (ib + NSC // TCB, sb, 0)),
            pl.BlockSpec((TCB, BS, D), lambda ib, sb: (ib + NSC // TCB, sb, 0)),
        ],
        out_specs=pl.BlockSpec((TCB, 1, 2 * D), lambda ib, sb: (ib, 0, 0)),
        out_shape=jax.ShapeDtypeStruct((nb_tc, 1, 2 * D), jnp.float32),
        scratch_shapes=[
            pltpu.VMEM((2 * TCB, D), jnp.float32),
            pltpu.SMEM((TCB,), jnp.float32),
            pltpu.SMEM((TCB,), jnp.float32),
        ],
    )(v, inputs_0, inputs_1)

    return jnp.concatenate([sc_out, tc_out.reshape(nb_tc, 2 * D)], axis=0)


# speedup vs baseline: 1.7756x; 1.0034x over previous
import math

import jax
import jax.numpy as jnp
from jax import lax
from jax.experimental import pallas as pl
from jax.experimental.pallas import tpu as pltpu
from jax.experimental.pallas import tpu_sc as plsc

D = 128
B = 16
S = 4096
NC = 2
NS = 16
L = 16
NJ = D // L

NSC = 16
BPC = NSC // 2
TPS = NS // BPC
R = S // TPS
CH = min(R, 128)
NCHUNK = R // CH
PCOLS = 2 * D + 128

NINF = -1e30


def _sc_body(in0_hbm, in1_hbm, w_hbm, fp_hbm, out_hbm,
             buf0, buf1, wmat, fp_ref, scores_ref, w_ref, stage, shared, part,
             fin, sem00, sem01, sem10, sem11):
    s_idx = lax.axis_index("s")
    c_idx = lax.axis_index("c")
    b = c_idx * BPC + s_idx // TPS
    p_idx = s_idx % TPS
    lane = lax.iota(jnp.int32, L)

    pltpu.sync_copy(w_hbm, wmat)
    pltpu.sync_copy(fp_hbm, fp_ref)
    inv_scale = 1.0 / math.sqrt(float(D))
    vs = []
    for j in range(NJ):
        diag_idx = lane + j * L
        dj = plsc.load_gather(wmat, [diag_idx, diag_idx])
        vs.append(dj * fp_ref[pl.ds(j * L, L)] * inv_scale)

    zeros = jnp.zeros((L,), jnp.float32)
    accs0 = tuple(zeros for _ in range(2 * NJ))
    row0 = p_idx * R
    sems0 = (sem00, sem01)
    sems1 = (sem10, sem11)

    def issue(c, slot):
        start = row0 + c * CH
        pltpu.async_copy(in1_hbm.at[b, pl.ds(start, CH)], buf1.at[slot],
                         sems1[slot])
        pltpu.async_copy(in0_hbm.at[b, pl.ds(start, CH)], buf0.at[slot],
                         sems0[slot])

    issue(0, 0)

    @pl.loop(0, NCHUNK // 2, init_carry=(jnp.float32(NINF), zeros, accs0))
    def chunk_loop(i, carry):
        m, l16, accs = carry
        for slot in range(2):
            c = 2 * i + slot

            @pl.when(c + 1 < NCHUNK)
            def _():
                issue(c + 1, 1 - slot)

            b1 = buf1.at[slot]
            b0 = buf0.at[slot]
            pltpu.make_async_copy(in1_hbm.at[b, pl.ds(row0, CH)],
                                  b1, sems1[slot]).wait()

            @pl.loop(0, CH // L)
            def score_groups(g):
                def row_score(r, sc16):
                    s = g * L + r
                    acc = vs[0] * b1[s, pl.ds(0, L)]
                    for j in range(1, NJ):
                        acc = acc + vs[j] * b1[s, pl.ds(j * L, L)]
                    sc = jnp.sum(acc)
                    return jnp.where(lane == r, sc, sc16)
                scores_ref[pl.ds(g * L, L)] = lax.fori_loop(
                    0, L, row_score, zeros, unroll=True)

            def group_max(g, m16):
                return jnp.maximum(m16, scores_ref[pl.ds(g * L, L)])
            m16 = lax.fori_loop(0, CH // L, group_max,
                                jnp.full((L,), NINF, jnp.float32),
                                unroll=4)
            mc = jnp.max(m16)
            m_new = jnp.maximum(m, mc)
            rs16 = jnp.exp(jnp.full((L,), m - m_new, jnp.float32))
            l16 = l16 * rs16
            accs = tuple(a * rs16 for a in accs)
            m_full = jnp.full((L,), m_new, jnp.float32)
            m = m_new

            def group_w(g, l16):
                w16 = jnp.exp(scores_ref[pl.ds(g * L, L)] - m_full)
                w_ref[pl.ds(g * L, L)] = w16
                return l16 + w16
            l16 = lax.fori_loop(0, CH // L, group_w, l16, unroll=4)

            pltpu.make_async_copy(in0_hbm.at[b, pl.ds(row0, CH)],
                                  b0, sems0[slot]).wait()

            def row_acc(s, accs):
                wspl = plsc.load_gather(
                    w_ref, [jnp.full((L,), s, jnp.int32)])
                new = []
                for j in range(NJ):
                    new.append(accs[j] + wspl * b0[s, pl.ds(j * L, L)])
                for j in range(NJ):
                    new.append(accs[NJ + j] + wspl * b1[s, pl.ds(j * L, L)])
                return tuple(new)
            accs = lax.fori_loop(0, CH, row_acc, accs, unroll=4)
        return m, l16, accs

    m, l16, accs = chunk_loop

    for j in range(2 * NJ):
        stage[pl.ds(j * L, L)] = accs[j]
    lsum = jnp.sum(l16)
    mf = jnp.full((L,), m, jnp.float32)
    lf = jnp.full((L,), lsum, jnp.float32)
    stage[pl.ds(2 * D, L)] = jnp.where(lane == 0, mf,
                                       jnp.where(lane == 1, lf, zeros))
    pltpu.sync_copy(stage, shared.at[s_idx])
    plsc.subcore_barrier()

    @pl.when(p_idx == 0)
    def _():
        gbase = s_idx
        pltpu.sync_copy(shared, part)
        macc = [jnp.full((L,), m, jnp.float32)]
        lacc = [jnp.full((L,), lsum, jnp.float32)]
        cur = [list(accs)]
        for t in range(1, TPS):
            row = gbase + t
            m_o = plsc.load_gather(
                part, [jnp.full((L,), row, jnp.int32),
                       jnp.full((L,), 2 * D, jnp.int32)])
            l_o = plsc.load_gather(
                part, [jnp.full((L,), row, jnp.int32),
                       jnp.full((L,), 2 * D + 1, jnp.int32)])
            mb = jnp.maximum(macc[0], m_o)
            cs = jnp.exp(macc[0] - mb)
            co = jnp.exp(m_o - mb)
            lacc[0] = lacc[0] * cs + l_o * co
            nxt = []
            for j in range(2 * NJ):
                acc_o = part[row, pl.ds(j * L, L)]
                nxt.append(cur[0][j] * cs + acc_o * co)
            cur[0] = nxt
            macc[0] = mb
        rinv = 1.0 / lacc[0]
        for j in range(2 * NJ):
            fin[pl.ds(j * L, L)] = cur[0][j] * rinv
        pltpu.sync_copy(fin, out_hbm.at[b])


@jax.jit
def kernel(inputs_0, inputs_1, W, b, fingerprint):
    mesh = plsc.VectorSubcoreMesh(
        core_axis_name="c", subcore_axis_name="s",
        num_cores=NC, num_subcores=NS)
    sc_call = pl.kernel(
        _sc_body,
        out_type=jax.ShapeDtypeStruct((NSC, 2 * D), jnp.float32),
        mesh=mesh,
        compiler_params=pltpu.CompilerParams(needs_layout_passes=False),
        scratch_types=[
            pltpu.VMEM((2, CH, D), jnp.float32),
            pltpu.VMEM((2, CH, D), jnp.float32),
            pltpu.VMEM((D, D), jnp.float32),
            pltpu.VMEM((D,), jnp.float32),
            pltpu.VMEM((CH,), jnp.float32),
            pltpu.VMEM((CH,), jnp.float32),
            pltpu.VMEM((PCOLS,), jnp.float32),
            pltpu.VMEM_SHARED((NS, PCOLS), jnp.float32),
            pltpu.VMEM((NS, PCOLS), jnp.float32),
            pltpu.VMEM((2 * D,), jnp.float32),
            pltpu.SemaphoreType.DMA,
            pltpu.SemaphoreType.DMA,
            pltpu.SemaphoreType.DMA,
            pltpu.SemaphoreType.DMA,
        ],
    )
    return sc_call(inputs_0, inputs_1, W, fingerprint)
